# Initial kernel scaffold; baseline (speedup 1.0000x reference)
import jax
import jax.numpy as jnp
from jax.experimental import pallas as pl


def kernel(x, incidence, mask, h_node, h_e, weight, bias, weight2, a):
    raise NotImplementedError("write your pallas kernel here")


# speedup vs baseline: 6.9525x; 6.9525x over previous
import functools

import jax
import jax.numpy as jnp
from jax.experimental import pallas as pl
from jax.experimental.pallas import tpu as pltpu

_ALPHA = 0.2
_EB = 256


def _lrelu(v):
    return jnp.where(v >= 0, v, _ALPHA * v)


def _bf(v):
    return v.astype(jnp.bfloat16).astype(jnp.float32)


def _stage1_body(R, C, xt_ref, mt_ref, hnb_ref, cst_ref, inc_ref,
                 out1_ref, sout_ref, s_scr):
    ge = pl.program_id(0)

    @pl.when(ge == 0)
    def _compute_s():
        wx = cst_ref[:, 0:1]
        wm = cst_ref[:, 1:2]
        w_pn = cst_ref[0:1, 2:3]
        for r in range(R):
            sr = (jnp.sum(_bf(xt_ref[r]) * wx, axis=0, keepdims=True)
                  + jnp.sum(_bf(mt_ref[r]) * wm, axis=0, keepdims=True)
                  + hnb_ref[0:1, :] * w_pn
                  + hnb_ref[1:2, :])
            s_scr[r:r + 1, :] = sr
        s_scr[R:R + 1, :] = jnp.ones_like(s_scr[R:R + 1, :])
        if R + 1 < 8:
            s_scr[R + 1:, :] = jnp.zeros_like(s_scr[R + 1:, :])
        sout_ref[:, :] = s_scr[:, :]

    out1_ref[:, :] = jnp.dot(s_scr[:, :], inc_ref[:, :],
                             preferred_element_type=jnp.float32)


def _stage2_body(R, out1_ref, st_ref, cst_ref, w2_ref, inc_ref,
                 edge_ref, node_ref):
    ge = pl.program_id(0)
    a0b = cst_ref[1:2, 2:3]
    a1b = cst_ref[2:3, 2:3]

    deg = out1_ref[R:R + 1, :]
    ep = out1_ref[:, :] * (1.0 / deg)
    edge_all = jnp.dot(ep, w2_ref[:, :],
                       preferred_element_type=jnp.float32)
    edge_ref[:, :] = edge_all

    @pl.when(ge == 0)
    def _init():
        node_ref[:, :] = jnp.zeros_like(node_ref)

    inc = inc_ref[:, :]
    n = inc.shape[0]
    cols = []
    for r in range(R):
        vq = _bf(st_ref[:, r:r + 1]) * a0b
        vy = _bf(edge_all[r:r + 1, :]) * a1b
        pe = _lrelu(vq + vy)
        mr = jnp.max(pe, axis=0, keepdims=True)
        attn_un = inc * jnp.exp(pe - mr)
        dr = jnp.sum(attn_un, axis=0, keepdims=True)
        wr = edge_all[r:r + 1, :] / dr
        cols.append(jax.lax.dot_general(
            attn_un, wr, (((1,), (1,)), ((), ())),
            preferred_element_type=jnp.float32))
    if R < 8:
        cols.append(jnp.zeros((n, 8 - R), jnp.float32))
    node_ref[:, :] += jnp.concatenate(cols, axis=1)


def kernel(x, incidence, mask, h_node, h_e, weight, bias, weight2, a):
    B, C, N, T = x.shape
    E = incidence.shape[1]
    R = B * T
    G = E // _EB

    f32 = jnp.float32
    bf = lambda v: v.astype(jnp.bfloat16).astype(f32)
    xt = jnp.transpose(x, (0, 3, 1, 2)).reshape(R, C, N)
    mt = jnp.transpose(mask, (0, 3, 1, 2)).reshape(R, C, N)
    hnb = jnp.concatenate(
        [bf(h_node), bias[None, :], jnp.zeros((6, N), f32)], axis=0)
    cst = (jnp.zeros((8, 128), f32)
           .at[:C, 0].set(bf(weight[:C, 0]))
           .at[:C, 1].set(bf(weight[C:2 * C, 0]))
           .at[0, 2].set(bf(weight[2 * C, 0]))
           .at[1, 2].set(bf(a[0, 0]))
           .at[2, 2].set(bf(a[1, 0])))

    out1, s_pad = pl.pallas_call(
        functools.partial(_stage1_body, R, C),
        grid=(G,),
        in_specs=[
            pl.BlockSpec((R, C, N), lambda i: (0, 0, 0)),
            pl.BlockSpec((R, C, N), lambda i: (0, 0, 0)),
            pl.BlockSpec((8, N), lambda i: (0, 0)),
            pl.BlockSpec((8, 128), lambda i: (0, 0)),
            pl.BlockSpec((N, _EB), lambda i: (0, i)),
        ],
        out_specs=[
            pl.BlockSpec((8, _EB), lambda i: (0, i)),
            pl.BlockSpec((8, N), lambda i: (0, 0)),
        ],
        out_shape=[
            jax.ShapeDtypeStruct((8, E), f32),
            jax.ShapeDtypeStruct((8, N), f32),
        ],
        scratch_shapes=[pltpu.VMEM((8, N), f32)],
    )(xt, mt, hnb, cst, incidence)

    st = s_pad.T

    edge_out, node_t = pl.pallas_call(
        functools.partial(_stage2_body, R),
        grid=(G,),
        in_specs=[
            pl.BlockSpec((8, E), lambda i: (0, 0)),
            pl.BlockSpec((N, 8), lambda i: (0, 0)),
            pl.BlockSpec((8, 128), lambda i: (0, 0)),
            pl.BlockSpec((E, _EB), lambda i: (0, i)),
            pl.BlockSpec((N, _EB), lambda i: (0, i)),
        ],
        out_specs=[
            pl.BlockSpec((8, _EB), lambda i: (0, i)),
            pl.BlockSpec((N, 8), lambda i: (0, 0)),
        ],
        out_shape=[
            jax.ShapeDtypeStruct((8, E), f32),
            jax.ShapeDtypeStruct((N, 8), f32),
        ],
    )(out1, st, cst, weight2, incidence)

    imputations = jnp.transpose(
        node_t[:, :R].reshape(N, B, T), (1, 0, 2))[:, None, :, :]
    edge_last = edge_out[T - 1:R:T, :][:, :, None]
    return imputations, edge_last
